# initial kernel scaffold (unmeasured)
import jax
import jax.numpy as jnp
from jax import lax
from jax.experimental import pallas as pl
from jax.experimental.pallas import tpu as pltpu

C = 256


def kernel(partial, resid, gamma):
    _, M, D = partial.shape
    HALF = M // 2
    NC = HALF // C
    assert HALF % C == 0 and NC >= 2

    def body(partial_ref, resid_ref, gamma_ref, out_ref, yrecv_ref,
             xsend, xrecv, ysend, ychunk, pchunk, rchunk, ochunk,
             xsend_sems, xrecv_sems, ysend_sems, yrecv_sems,
             pin_sems, rin_sems, yin_sems, oout_sems):
        my_x = lax.axis_index("x")
        my_y = lax.axis_index("y")
        xnbr = (1 - my_x, my_y)
        ynbr = (my_x, 1 - my_y)
        row0 = my_y * HALF

        barrier = pltpu.get_barrier_semaphore()
        for nbr in (xnbr, ynbr):
            pl.semaphore_signal(barrier, inc=1, device_id=nbr,
                                device_id_type=pl.DeviceIdType.MESH)
        pl.semaphore_wait(barrier, 2)

        def xcopy(k):
            return pltpu.make_async_remote_copy(
                src_ref=xsend.at[k % 2],
                dst_ref=xrecv.at[k],
                send_sem=xsend_sems.at[k % 2],
                recv_sem=xrecv_sems.at[k],
                device_id=xnbr,
                device_id_type=pl.DeviceIdType.MESH,
            )

        def ycopy(k):
            return pltpu.make_async_remote_copy(
                src_ref=ysend.at[k % 2],
                dst_ref=yrecv_ref.at[pl.ds(k * C, C), :],
                send_sem=ysend_sems.at[k % 2],
                recv_sem=yrecv_sems.at[k],
                device_id=ynbr,
                device_id_type=pl.DeviceIdType.MESH,
            )

        def pload(k):
            return pltpu.make_async_copy(
                partial_ref.at[0, pl.ds(row0 + k * C, C), :],
                pchunk.at[k % 2], pin_sems.at[k % 2])

        def rload(k):
            return pltpu.make_async_copy(
                resid_ref.at[pl.ds(row0 + k * C, C), :],
                rchunk.at[k % 2], rin_sems.at[k % 2])

        def yload(k):
            return pltpu.make_async_copy(
                yrecv_ref.at[pl.ds(k * C, C), :],
                ychunk.at[k % 2], yin_sems.at[k % 2])

        def ostore(k, dst_row):
            return pltpu.make_async_copy(
                ochunk.at[k % 2],
                out_ref.at[pl.ds(dst_row, C), :],
                oout_sems.at[k % 2])

        pload(0).start()
        for k in range(NC):
            pload(k).wait()
            if k + 1 < NC:
                pload(k + 1).start()
            if k >= 2:
                xcopy(k - 2).wait_send()
            xsend[k % 2] = pchunk[k % 2].astype(jnp.bfloat16)
            xcopy(k).start()

        pload(0).start()
        rload(0).start()
        pending_ostore = [None, None]
        for k in range(NC):
            slot = k % 2
            pload(k).wait()
            rload(k).wait()
            if k + 1 < NC:
                pload(k + 1).start()
                rload(k + 1).start()
            xcopy(k).wait_recv()
            y32 = pchunk[slot] + xrecv[k].astype(jnp.float32) + rchunk[slot]
            ms = jnp.mean(y32 * y32, axis=-1, keepdims=True)
            o = y32 * lax.rsqrt(ms + 1e-6) * gamma_ref[:]
            if k >= 2:
                pending_ostore[slot].wait()
                ycopy(k - 2).wait_send()
            ochunk[slot] = o
            st = ostore(k, row0 + k * C)
            st.start()
            pending_ostore[slot] = st
            ysend[slot] = o.astype(jnp.bfloat16)
            ycopy(k).start()

        base = (1 - my_y) * HALF
        for k in range(NC):
            slot = k % 2
            ycopy(k).wait_recv()
            yload(k).start()
            yload(k).wait()
            pending_ostore[slot].wait()
            ochunk[slot] = ychunk[slot].astype(jnp.float32)
            st = ostore(k, base + k * C)
            st.start()
            pending_ostore[slot] = st

        pending_ostore[0].wait()
        pending_ostore[1].wait()
        for k in (NC - 2, NC - 1):
            xcopy(k).wait_send()
            ycopy(k).wait_send()

    out = pl.pallas_call(
        body,
        out_shape=[
            jax.ShapeDtypeStruct((M, D), jnp.float32),
            jax.ShapeDtypeStruct((HALF, D), jnp.bfloat16),
        ],
        in_specs=[
            pl.BlockSpec(memory_space=pltpu.ANY),
            pl.BlockSpec(memory_space=pltpu.ANY),
            pl.BlockSpec(memory_space=pltpu.VMEM),
        ],
        out_specs=[
            pl.BlockSpec(memory_space=pltpu.ANY),
            pl.BlockSpec(memory_space=pltpu.ANY),
        ],
        scratch_shapes=[
            pltpu.VMEM((2, C, D), jnp.bfloat16),
            pltpu.VMEM((NC, C, D), jnp.bfloat16),
            pltpu.VMEM((2, C, D), jnp.bfloat16),
            pltpu.VMEM((2, C, D), jnp.bfloat16),
            pltpu.VMEM((2, C, D), jnp.float32),
            pltpu.VMEM((2, C, D), jnp.float32),
            pltpu.VMEM((2, C, D), jnp.float32),
            pltpu.SemaphoreType.DMA((2,)),
            pltpu.SemaphoreType.DMA((NC,)),
            pltpu.SemaphoreType.DMA((2,)),
            pltpu.SemaphoreType.DMA((NC,)),
            pltpu.SemaphoreType.DMA((2,)),
            pltpu.SemaphoreType.DMA((2,)),
            pltpu.SemaphoreType.DMA((2,)),
            pltpu.SemaphoreType.DMA((2,)),
        ],
        compiler_params=pltpu.CompilerParams(collective_id=0),
    )(partial, resid, gamma.reshape(1, D))
    return out[0]


# baseline (device time: 376537 ns/iter reference)
import jax
import jax.numpy as jnp
from jax import lax
from jax.experimental import pallas as pl
from jax.experimental.pallas import tpu as pltpu

C = 256


def kernel(partial, resid, gamma):
    _, M, D = partial.shape
    HALF = M // 2
    NC = HALF // C
    assert HALF % C == 0 and NC >= 2

    def body(partial_ref, resid_ref, gamma_ref, out_ref, yrecv_ref,
             xsend, xrecv, ysend, ychunk, pchunk, rchunk, ochunk,
             xsend_sems, xrecv_sems, ysend_sems, yrecv_sems,
             pin_sems, rin_sems, yin_sems, oout_sems):
        my_x = lax.axis_index("x")
        my_y = lax.axis_index("y")
        xnbr = (1 - my_x, my_y)
        ynbr = (my_x, 1 - my_y)
        row0 = my_y * HALF

        barrier = pltpu.get_barrier_semaphore()
        for nbr in (xnbr, ynbr):
            pl.semaphore_signal(barrier, inc=1, device_id=nbr,
                                device_id_type=pl.DeviceIdType.MESH)
        pl.semaphore_wait(barrier, 2)

        def xcopy(k):
            return pltpu.make_async_remote_copy(
                src_ref=xsend.at[k % 2],
                dst_ref=xrecv.at[k],
                send_sem=xsend_sems.at[k % 2],
                recv_sem=xrecv_sems.at[k],
                device_id=xnbr,
                device_id_type=pl.DeviceIdType.MESH,
            )

        def ycopy(k):
            return pltpu.make_async_remote_copy(
                src_ref=ysend.at[k % 2],
                dst_ref=yrecv_ref.at[pl.ds(k * C, C), :],
                send_sem=ysend_sems.at[k % 2],
                recv_sem=yrecv_sems.at[k],
                device_id=ynbr,
                device_id_type=pl.DeviceIdType.MESH,
            )

        def pload(k):
            return pltpu.make_async_copy(
                partial_ref.at[pl.ds(row0 + k * C, C), :],
                pchunk.at[k % 2], pin_sems.at[k % 2])

        def rload(k):
            return pltpu.make_async_copy(
                resid_ref.at[pl.ds(row0 + k * C, C), :],
                rchunk.at[k % 2], rin_sems.at[k % 2])

        def yload(k):
            return pltpu.make_async_copy(
                yrecv_ref.at[pl.ds(k * C, C), :],
                ychunk.at[k % 2], yin_sems.at[k % 2])

        def ostore(k, dst_row):
            return pltpu.make_async_copy(
                ochunk.at[k % 2],
                out_ref.at[pl.ds(dst_row, C), :],
                oout_sems.at[k % 2])

        with jax.named_scope("phase1_send"):
            pload(0).start()
            for k in range(NC):
                pload(k).wait()
                if k + 1 < NC:
                    pload(k + 1).start()
                if k >= 2:
                    xcopy(k - 2).wait_send()
                xsend[k % 2] = pchunk[k % 2].astype(jnp.bfloat16)
                xcopy(k).start()

        pload(0).start()
        rload(0).start()
        pending_ostore = [None, None]
        for k in range(NC):
            slot = k % 2
            with jax.named_scope(f"p2_load#k={k}"):
                pload(k).wait()
                rload(k).wait()
                if k + 1 < NC:
                    pload(k + 1).start()
                    rload(k + 1).start()
            with jax.named_scope(f"p2_xwait#k={k}"):
                xcopy(k).wait_recv()
            with jax.named_scope(f"p2_compute#k={k}"):
                y32 = pchunk[slot] + xrecv[k].astype(jnp.float32) + rchunk[slot]
                ms = jnp.mean(y32 * y32, axis=-1, keepdims=True)
                o = y32 * lax.rsqrt(ms + 1e-6) * gamma_ref[:]
                if k >= 2:
                    pending_ostore[slot].wait()
                    ycopy(k - 2).wait_send()
                ochunk[slot] = o
                st = ostore(k, row0 + k * C)
                st.start()
                pending_ostore[slot] = st
                ysend[slot] = o.astype(jnp.bfloat16)
                ycopy(k).start()

        base = (1 - my_y) * HALF
        for k in range(NC):
            slot = k % 2
            with jax.named_scope(f"p3_ywait#k={k}"):
                ycopy(k).wait_recv()
            with jax.named_scope(f"p3_store#k={k}"):
                yload(k).start()
                yload(k).wait()
                pending_ostore[slot].wait()
                ochunk[slot] = ychunk[slot].astype(jnp.float32)
                st = ostore(k, base + k * C)
                st.start()
                pending_ostore[slot] = st

        pending_ostore[0].wait()
        pending_ostore[1].wait()
        for k in (NC - 2, NC - 1):
            xcopy(k).wait_send()
            ycopy(k).wait_send()

    out = pl.pallas_call(
        body,
        out_shape=[
            jax.ShapeDtypeStruct((M, D), jnp.float32),
            jax.ShapeDtypeStruct((HALF, D), jnp.bfloat16),
        ],
        in_specs=[
            pl.BlockSpec(memory_space=pltpu.ANY),
            pl.BlockSpec(memory_space=pltpu.ANY),
            pl.BlockSpec(memory_space=pltpu.VMEM),
        ],
        out_specs=[
            pl.BlockSpec(memory_space=pltpu.ANY),
            pl.BlockSpec(memory_space=pltpu.ANY),
        ],
        scratch_shapes=[
            pltpu.VMEM((2, C, D), jnp.bfloat16),
            pltpu.VMEM((NC, C, D), jnp.bfloat16),
            pltpu.VMEM((2, C, D), jnp.bfloat16),
            pltpu.VMEM((2, C, D), jnp.bfloat16),
            pltpu.VMEM((2, C, D), jnp.float32),
            pltpu.VMEM((2, C, D), jnp.float32),
            pltpu.VMEM((2, C, D), jnp.float32),
            pltpu.SemaphoreType.DMA((2,)),
            pltpu.SemaphoreType.DMA((NC,)),
            pltpu.SemaphoreType.DMA((2,)),
            pltpu.SemaphoreType.DMA((NC,)),
            pltpu.SemaphoreType.DMA((2,)),
            pltpu.SemaphoreType.DMA((2,)),
            pltpu.SemaphoreType.DMA((2,)),
            pltpu.SemaphoreType.DMA((2,)),
        ],
        compiler_params=pltpu.CompilerParams(
            collective_id=0, vmem_limit_bytes=62 * 1024 * 1024),
    )(partial.reshape(M, D), resid, gamma.reshape(1, D))
    return out[0]


# device time: 261093 ns/iter; 1.4422x vs baseline; 1.4422x over previous
import jax
import jax.numpy as jnp
from jax import lax
from jax.experimental import pallas as pl
from jax.experimental.pallas import tpu as pltpu

C = 256


def kernel(partial, resid, gamma):
    _, M, D = partial.shape
    HALF = M // 2
    NC = HALF // C
    assert HALF % C == 0 and NC >= 4

    def body(partial_ref, resid_ref, gamma_ref, out_ref, yrecv_ref,
             xsend, xrecv, ysend, ychunk, pchunk, rchunk, ochunk,
             xsend_sems, xrecv_sems, ysend_sems, yrecv_sems,
             pin_sems, rin_sems, yin_sems, oout_sems):
        my_x = lax.axis_index("x")
        my_y = lax.axis_index("y")
        xnbr = (1 - my_x, my_y)
        ynbr = (my_x, 1 - my_y)
        row0 = my_y * HALF
        base = (1 - my_y) * HALF

        barrier = pltpu.get_barrier_semaphore()
        for nbr in (xnbr, ynbr):
            pl.semaphore_signal(barrier, inc=1, device_id=nbr,
                                device_id_type=pl.DeviceIdType.MESH)
        pl.semaphore_wait(barrier, 2)

        def xcopy(k):
            return pltpu.make_async_remote_copy(
                src_ref=xsend.at[k % 2],
                dst_ref=xrecv.at[k],
                send_sem=xsend_sems.at[k % 2],
                recv_sem=xrecv_sems.at[k],
                device_id=xnbr,
                device_id_type=pl.DeviceIdType.MESH,
            )

        def ycopy(k):
            return pltpu.make_async_remote_copy(
                src_ref=ysend.at[k % 2],
                dst_ref=yrecv_ref.at[pl.ds(k * C, C), :],
                send_sem=ysend_sems.at[k % 2],
                recv_sem=yrecv_sems.at[k],
                device_id=ynbr,
                device_id_type=pl.DeviceIdType.MESH,
            )

        def pload(k):
            return pltpu.make_async_copy(
                partial_ref.at[pl.ds(row0 + k * C, C), :],
                pchunk.at[k % 3], pin_sems.at[k % 3])

        def rload(k):
            return pltpu.make_async_copy(
                resid_ref.at[pl.ds(row0 + k * C, C), :],
                rchunk.at[k % 2], rin_sems.at[k % 2])

        def yload(k):
            return pltpu.make_async_copy(
                yrecv_ref.at[pl.ds(k * C, C), :],
                ychunk.at[0], yin_sems.at[0])

        pending_ostore = [None, None, None]
        store_idx = [0]

        def store(value, dst_row):
            slot = store_idx[0] % 3
            if pending_ostore[slot] is not None:
                pending_ostore[slot].wait()
            ochunk[slot] = value
            st = pltpu.make_async_copy(
                ochunk.at[slot],
                out_ref.at[pl.ds(dst_row, C), :],
                oout_sems.at[slot])
            st.start()
            pending_ostore[slot] = st
            store_idx[0] += 1

        pload(0).start()
        rload(0).start()

        for k in range(NC + 3):
            if k < NC:
                pload(k).wait()
                if k >= 2:
                    xcopy(k - 2).wait_send()
                xsend[k % 2] = pchunk[k % 3].astype(jnp.bfloat16)
                xcopy(k).start()
                if k + 1 < NC:
                    pload(k + 1).start()

            j = k - 1
            if 0 <= j < NC:
                rload(j).wait()
                if j + 1 < NC:
                    rload(j + 1).start()
                xcopy(j).wait_recv()
                y32 = (pchunk[j % 3]
                       + xrecv[j].astype(jnp.float32) + rchunk[j % 2])
                ms = jnp.mean(y32 * y32, axis=-1, keepdims=True)
                o = y32 * lax.rsqrt(ms + 1e-6) * gamma_ref[:]
                if j >= 2:
                    ycopy(j - 2).wait_send()
                ysend[j % 2] = o.astype(jnp.bfloat16)
                ycopy(j).start()
                store(o, row0 + j * C)

            j2 = k - 3
            if 0 <= j2 < NC:
                ycopy(j2).wait_recv()
                yload(j2).start()
                yload(j2).wait()
                store(ychunk[0].astype(jnp.float32), base + j2 * C)

        for st in pending_ostore:
            if st is not None:
                st.wait()
        for k in (NC - 2, NC - 1):
            xcopy(k).wait_send()
            ycopy(k).wait_send()

    out = pl.pallas_call(
        body,
        out_shape=[
            jax.ShapeDtypeStruct((M, D), jnp.float32),
            jax.ShapeDtypeStruct((HALF, D), jnp.bfloat16),
        ],
        in_specs=[
            pl.BlockSpec(memory_space=pl.ANY),
            pl.BlockSpec(memory_space=pl.ANY),
            pl.BlockSpec(memory_space=pltpu.VMEM),
        ],
        out_specs=[
            pl.BlockSpec(memory_space=pl.ANY),
            pl.BlockSpec(memory_space=pl.ANY),
        ],
        scratch_shapes=[
            pltpu.VMEM((2, C, D), jnp.bfloat16),
            pltpu.VMEM((NC, C, D), jnp.bfloat16),
            pltpu.VMEM((2, C, D), jnp.bfloat16),
            pltpu.VMEM((1, C, D), jnp.bfloat16),
            pltpu.VMEM((3, C, D), jnp.float32),
            pltpu.VMEM((2, C, D), jnp.float32),
            pltpu.VMEM((3, C, D), jnp.float32),
            pltpu.SemaphoreType.DMA((2,)),
            pltpu.SemaphoreType.DMA((NC,)),
            pltpu.SemaphoreType.DMA((2,)),
            pltpu.SemaphoreType.DMA((NC,)),
            pltpu.SemaphoreType.DMA((3,)),
            pltpu.SemaphoreType.DMA((2,)),
            pltpu.SemaphoreType.DMA((1,)),
            pltpu.SemaphoreType.DMA((3,)),
        ],
        compiler_params=pltpu.CompilerParams(
            collective_id=0, vmem_limit_bytes=63 * 1024 * 1024),
    )(partial.reshape(M, D), resid, gamma.reshape(1, D))
    return out[0]
